# baseline (speedup 1.0000x reference)
import jax
import jax.numpy as jnp
from jax.experimental import pallas as pl
from jax.experimental.pallas import tpu as pltpu

TOTAL = 32768
BATCH = 16
VAL = 2 * TOTAL


def _body(a_ref, aoff_ref, b_ref, boff_ref, id_ref,
          vals_ref, lens_ref, offs_ref):
    vals_ref[pl.ds(0, TOTAL)] = a_ref[...]
    vals_ref[pl.ds(TOTAL, TOTAL)] = b_ref[...]
    vals_ref[pl.ds(VAL, BATCH)] = id_ref[...]

    aoff = aoff_ref[...]
    boff = boff_ref[...]
    a_lo = aoff[0:BATCH]
    a_hi = aoff[1:BATCH + 1]
    b_lo = boff[0:BATCH]
    b_hi = boff[1:BATCH + 1]
    ramp = jax.lax.broadcasted_iota(jnp.int32, (BATCH,), 0)
    lens_ref[...] = jnp.concatenate(
        [a_hi - a_lo, b_hi - b_lo, jnp.ones((BATCH,), jnp.int32)])
    offs_ref[...] = jnp.concatenate(
        [aoff, b_hi + TOTAL, ramp + (VAL + 1)])


def kernel(feat_a__values, feat_a__offsets, feat_b__values, feat_b__offsets, id):
    out = pl.pallas_call(
        _body,
        out_shape=(
            jax.ShapeDtypeStruct((VAL + BATCH,), jnp.float32),
            jax.ShapeDtypeStruct((3 * BATCH,), jnp.int32),
            jax.ShapeDtypeStruct((3 * BATCH + 1,), jnp.int32),
        ),
    )(feat_a__values, feat_a__offsets, feat_b__values, feat_b__offsets, id)
    return tuple(out)


# speedup vs baseline: 1.0016x; 1.0016x over previous
import jax
import jax.numpy as jnp
from jax.experimental import pallas as pl

TOTAL = 32768
BATCH = 16
VAL = 2 * TOTAL


def _body(a_ref, aoff_ref, b_ref, boff_ref, id_ref,
          vals_ref, lens_ref, offs_ref):
    vals_ref[pl.ds(0, TOTAL)] = a_ref[...]
    vals_ref[pl.ds(TOTAL, TOTAL)] = b_ref[...]
    vals_ref[pl.ds(VAL, BATCH)] = id_ref[...]

    aoff = aoff_ref[...]
    boff = boff_ref[...]
    a_lo = aoff[0:BATCH]
    a_hi = aoff[1:BATCH + 1]
    b_lo = boff[0:BATCH]
    b_hi = boff[1:BATCH + 1]
    ramp = jax.lax.broadcasted_iota(jnp.int32, (BATCH,), 0)
    lens_ref[...] = jnp.concatenate(
        [a_hi - a_lo, b_hi - b_lo, jnp.ones((BATCH,), jnp.int32)])
    offs_ref[...] = jnp.concatenate(
        [aoff, b_hi + TOTAL, ramp + (VAL + 1)])


def kernel(feat_a__values, feat_a__offsets, feat_b__values, feat_b__offsets, id):
    out = pl.pallas_call(
        _body,
        out_shape=(
            jax.ShapeDtypeStruct((VAL + BATCH,), jnp.float32),
            jax.ShapeDtypeStruct((3 * BATCH,), jnp.int32),
            jax.ShapeDtypeStruct((3 * BATCH + 1,), jnp.int32),
        ),
    )(feat_a__values, feat_a__offsets, feat_b__values, feat_b__offsets, id)
    return tuple(out)
